# baseline (speedup 1.0000x reference)
import math

import jax
import jax.numpy as jnp
from jax import lax
from jax.experimental import pallas as pl
from jax.experimental.pallas import tpu as pltpu
from jax.experimental.pallas import tpu_sc as plsc

D = 1024
B = 4
S = 2048
SCALE = math.sqrt(D)

NC = 2
NS = 16
LANES = 16
NW = NC * NS
SPW = S // NW
CHUNK = 8
NT = SPW // CHUNK
ROWS = B * CHUNK
NBUF = 3
AHEAD = 2


def _body(x_hbm, pe_hbm, table_hbm, out_hbm, idx_v, r0, r1, r2, p0, p1, p2,
          g0, g1, g2, ps0, ps1, ps2, s0_, s1_, s2_):
    rows = (r0, r1, r2)
    pes = (p0, p1, p2)
    gsem = (g0, g1, g2)
    psem = (ps0, ps1, ps2)
    ssem = (s0_, s1_, s2_)
    wid = lax.axis_index("s") * NC + lax.axis_index("c")
    s_base = wid * SPW

    for b in range(B):
        pltpu.sync_copy(x_hbm.at[b, pl.ds(s_base, SPW)], idx_v.at[b])

    gather_d = [[None] * B for _ in range(NT)]
    pe_d = [None] * NT
    store_d = [[None] * B for _ in range(NT)]

    def fire(t):
        k = t % NBUF
        pe_d[t] = pltpu.async_copy(
            pe_hbm.at[pl.ds(s_base + t * CHUNK, CHUNK)], pes[k], psem[k])
        for b in range(B):
            gather_d[t][b] = pltpu.async_copy(
                table_hbm.at[idx_v.at[b, pl.ds(t * CHUNK, CHUNK)]],
                rows[k].at[pl.ds(b * CHUNK, CHUNK)], gsem[k])

    for t in range(AHEAD):
        fire(t)

    for t in range(NT):
        k = t % NBUF
        buf = rows[k]
        pe_b = pes[k]
        pe_d[t].wait()
        for b in range(B):
            gather_d[t][b].wait()

        JU = 8

        def fma_row(r, _):
            def jblk(j8, _2):
                base = j8 * (JU * LANES)
                for jj in range(JU):
                    sl = pl.ds(base + jj * LANES, LANES)
                    pv = pe_b[r, sl]
                    for b in range(B):
                        buf[b * CHUNK + r, sl] = (
                            buf[b * CHUNK + r, sl] * SCALE + pv)
                return 0

            lax.fori_loop(0, D // LANES // JU, jblk, 0)
            return 0

        lax.fori_loop(0, CHUNK, fma_row, 0)

        if t + AHEAD < NT:
            if t >= 1:
                for b in range(B):
                    store_d[t - 1][b].wait()
            fire(t + AHEAD)
        for b in range(B):
            store_d[t][b] = pltpu.async_copy(
                buf.at[pl.ds(b * CHUNK, CHUNK)],
                out_hbm.at[pl.ds(b * S + s_base + t * CHUNK, CHUNK)],
                ssem[k])

    for t in range(NT - AHEAD - 1, NT):
        if t >= 0:
            for b in range(B):
                if store_d[t][b] is not None:
                    store_d[t][b].wait()


@jax.jit
def _embed(x, table, pe2d):
    mesh = plsc.VectorSubcoreMesh(core_axis_name="c", subcore_axis_name="s")
    return pl.kernel(
        _body,
        out_type=jax.ShapeDtypeStruct((B * S, D), jnp.float32),
        mesh=mesh,
        scratch_types=(
            [pltpu.VMEM((B, SPW), jnp.int32)]
            + [pltpu.VMEM((ROWS, D), jnp.float32) for _ in range(NBUF)]
            + [pltpu.VMEM((CHUNK, D), jnp.float32) for _ in range(NBUF)]
            + [pltpu.SemaphoreType.DMA for _ in range(3 * NBUF)]
        ),
    )(x, pe2d, table)


def kernel(x, table, pe):
    pe2d = pe.reshape(pe.shape[1], D)
    out = _embed(x, table, pe2d)
    return out.reshape(B, S, D)
